# baseline (speedup 1.0000x reference)
import jax
import jax.numpy as jnp
from jax.experimental import pallas as pl

_DEF = jax.lax.Precision.DEFAULT
_HI = jax.lax.Precision.HIGHEST


def _tome_body(kl_ref, hid_ref, ts_ref, out_ref, tso_ref):
    m = jnp.mean(kl_ref[...], axis=0)
    m = m / jnp.sqrt(jnp.sum(m * m, axis=1, keepdims=True))
    ci = jax.lax.broadcasted_iota(jnp.int32, (1024, 2048), 1)
    rk = jax.lax.broadcasted_iota(jnp.int32, (1024, 1), 0)
    so = jnp.where(ci == 2 * rk + 1, 1.0, 0.0)
    b = jax.lax.dot_general(so, m, (((1,), (0,)), ((), ())),
                            precision=_HI)
    scores = jax.lax.dot_general(m, b, (((1,), (1,)), ((), ())),
                                 precision=_DEF)
    node_max = jnp.max(scores, axis=1, keepdims=True)
    ck = jax.lax.broadcasted_iota(jnp.int32, (2048, 1024), 1)
    nidx = jnp.min(jnp.where(scores == node_max, ck, 1024),
                   axis=1, keepdims=True)
    ri = jax.lax.broadcasted_iota(jnp.int32, (2048, 1), 0)
    evenok = (ri % 2 == 0) & (ri >= 2)
    merge = evenok & (ck == nidx)
    is_dst = ri == 2 * ck + 1
    mm = jnp.where(merge | is_dst, 1.0, 0.0)
    y = jax.lax.dot_general(mm, hid_ref[...], (((0,), (0,)), ((), ())),
                            precision=_DEF)
    ones = jnp.ones((2048, 1), dtype=jnp.float32)
    cnt1 = jax.lax.dot_general(mm, ones, (((0,), (0,)), ((), ())),
                               precision=_DEF)
    ts_y = jax.lax.dot_general(mm, ts_ref[...], (((0,), (0,)), ((), ())),
                               precision=_DEF)
    out_ref[0:1, :] = hid_ref[0:1, :]
    out_ref[pl.ds(1, 1024), :] = y / cnt1
    tso_ref[0:1, :] = ts_ref[0:1, :]
    tso_ref[pl.ds(1, 1024), :] = ts_y


def kernel(hidden_states, attention_mask, self_attention_scores, key_layer,
           tome_size):
    del attention_mask, self_attention_scores
    out, ts_out = pl.pallas_call(
        _tome_body,
        out_shape=(
            jax.ShapeDtypeStruct((1025, 768), jnp.float32),
            jax.ShapeDtypeStruct((1025, 1), jnp.float32),
        ),
    )(key_layer[0], hidden_states[0], tome_size[0])

    preserved = out[None]
    new_ts = ts_out[None]
    mask = jnp.zeros((1, 1, 1, 1025), dtype=hidden_states.dtype)
    return preserved, mask, new_ts


# speedup vs baseline: 1.5091x; 1.5091x over previous
import jax
import jax.numpy as jnp
from jax.experimental import pallas as pl

_DEF = jax.lax.Precision.DEFAULT
_HI = jax.lax.Precision.HIGHEST


def _tome_body(kl_ref, hid_ref, ts_ref, out_ref, tso_ref):
    mt = jnp.mean(kl_ref[...], axis=0)
    m = jnp.transpose(mt, (1, 0))
    m = m / jnp.sqrt(jnp.sum(m * m, axis=1, keepdims=True))
    ci = jax.lax.broadcasted_iota(jnp.int32, (1024, 2048), 1)
    rk = jax.lax.broadcasted_iota(jnp.int32, (1024, 1), 0)
    so = jnp.where(ci == 2 * rk + 1, 1.0, 0.0)
    b = jax.lax.dot_general(so, m, (((1,), (0,)), ((), ())),
                            precision=_HI)
    scores = jax.lax.dot_general(m, b, (((1,), (1,)), ((), ())),
                                 precision=_DEF)
    node_max = jnp.max(scores, axis=1, keepdims=True)
    ck = jax.lax.broadcasted_iota(jnp.int32, (2048, 1024), 1)
    nidx = jnp.min(jnp.where(scores == node_max, ck, 1024),
                   axis=1, keepdims=True)
    ri = jax.lax.broadcasted_iota(jnp.int32, (2048, 1), 0)
    evenok = (ri % 2 == 0) & (ri >= 2)
    merge = evenok & (ck == nidx)
    is_dst = ri == 2 * ck + 1
    mm = jnp.where(merge | is_dst, 1.0, 0.0)
    y = jax.lax.dot_general(mm, hid_ref[...], (((0,), (0,)), ((), ())),
                            precision=_DEF)
    ones = jnp.ones((2048, 1), dtype=jnp.float32)
    cnt1 = jax.lax.dot_general(mm, ones, (((0,), (0,)), ((), ())),
                               precision=_DEF)
    ts_y = jax.lax.dot_general(mm, ts_ref[...], (((0,), (0,)), ((), ())),
                               precision=_DEF)
    out_ref[0:1, :] = hid_ref[0:1, :]
    out_ref[pl.ds(1, 1024), :] = y / cnt1
    tso_ref[0:1, :] = ts_ref[0:1, :]
    tso_ref[pl.ds(1, 1024), :] = ts_y


def kernel(hidden_states, attention_mask, self_attention_scores, key_layer,
           tome_size):
    del attention_mask, self_attention_scores
    out, ts_out = pl.pallas_call(
        _tome_body,
        out_shape=(
            jax.ShapeDtypeStruct((1025, 768), jnp.float32),
            jax.ShapeDtypeStruct((1025, 1), jnp.float32),
        ),
    )(jnp.transpose(key_layer[0], (0, 2, 1)), hidden_states[0], tome_size[0])

    preserved = out[None]
    new_ts = ts_out[None]
    mask = jnp.zeros((1, 1, 1, 1025), dtype=hidden_states.dtype)
    return preserved, mask, new_ts
